# baseline (speedup 1.0000x reference)
import functools

import jax
import jax.numpy as jnp
from jax import lax
from jax.experimental import pallas as pl
from jax.experimental.pallas import tpu as pltpu
from jax.experimental.pallas import tpu_sc as plsc

N = 10000
E = 320000
D = 128

NC = 2
NS = 16
NW = NC * NS

NPAD = 10240
RPW = NPAD // NS
CH = 128
NCHUNK = 80
TOTCH = NCHUNK * NW
EPAD = TOTCH * CH

ROW_BLK = 2560


def _sc_agg_body(with_deg, y_hbm, eidx_hbm, z2d_hbm, z1d_hbm,
                 parts_hbm, deg_hbm, acc, ibuf, rows, deg_l, sems):
    c = lax.axis_index("c")
    s = lax.axis_index("s")
    wid = s * NC + c
    gsem = sems[0]
    base = wid * NCHUNK

    ones = jnp.full((16,), 1.0, jnp.float32)

    pltpu.sync_copy(z2d_hbm.at[pl.ds(s * RPW, RPW)], acc.at[pl.ds(s * RPW, RPW)])
    if with_deg:
        pltpu.sync_copy(z1d_hbm.at[:], deg_l)
    plsc.subcore_barrier()

    def chunk(g, carry):
        pltpu.sync_copy(eidx_hbm.at[base + g], ibuf)
        pltpu.async_copy(y_hbm.at[ibuf.at[0]], rows, gsem).wait()
        pltpu.sync_copy(rows, acc.at[ibuf.at[1]], add=True)
        if with_deg:
            for j in range(CH // 16):
                idx16 = ibuf[1, pl.ds(j * 16, 16)]
                plsc.addupdate_scatter(deg_l, [idx16], ones)
        return carry

    lax.fori_loop(0, NCHUNK, chunk, 0)
    plsc.subcore_barrier()

    pltpu.sync_copy(acc.at[pl.ds(s * RPW, RPW)],
                    parts_hbm.at[c, pl.ds(s * RPW, RPW)])
    if with_deg:
        pltpu.sync_copy(deg_l, deg_hbm.at[wid])


def _make_sc_agg(with_deg):
    mesh = plsc.VectorSubcoreMesh(core_axis_name="c", subcore_axis_name="s",
                                  num_cores=NC, num_subcores=NS)
    out_type = [jax.ShapeDtypeStruct((NC, NPAD, D), jnp.float32)]
    if with_deg:
        out_type.append(jax.ShapeDtypeStruct((NW, NPAD), jnp.float32))
    scratch = [
        pltpu.VMEM_SHARED((NPAD, D), jnp.float32),
        pltpu.VMEM((2, CH), jnp.int32),
        pltpu.VMEM((CH, D), jnp.float32),
    ]
    if with_deg:
        scratch.append(pltpu.VMEM((NPAD,), jnp.float32))
    scratch += [pltpu.SemaphoreType.DMA]

    def body(y_hbm, eidx_hbm, z2d_hbm, z1d_hbm, *rest):
        if with_deg:
            parts_hbm, deg_hbm = rest[0], rest[1]
            acc, ibuf, rows, deg_l = rest[2:6]
            sems = rest[6:]
        else:
            parts_hbm, deg_hbm = rest[0], None
            acc, ibuf, rows = rest[1:4]
            deg_l = None
            sems = rest[4:]
        _sc_agg_body(with_deg, y_hbm, eidx_hbm, z2d_hbm, z1d_hbm,
                     parts_hbm, deg_hbm, acc, ibuf, rows, deg_l, sems)

    return pl.kernel(body, out_type=out_type, mesh=mesh, scratch_types=scratch,
                     compiler_params=pltpu.CompilerParams(
                         needs_layout_passes=False),
                     name="sc_agg_deg" if with_deg else "sc_agg")


_make_sc_agg = functools.lru_cache(maxsize=None)(_make_sc_agg)


def _tc_transform_body(x_ref, w_ref, b_ref, y_ref, z_ref):
    out = jnp.dot(x_ref[...], w_ref[...],
                  preferred_element_type=jnp.float32,
                  precision=lax.Precision.HIGHEST)
    y_ref[...] = out[:, :D]
    z_ref[...] = out[:, D:] + b_ref[0:1, :]


_tc_transform = pl.pallas_call(
    _tc_transform_body,
    grid=(NPAD // ROW_BLK,),
    in_specs=[
        pl.BlockSpec((ROW_BLK, D), lambda i: (i, 0)),
        pl.BlockSpec((D, 2 * D), lambda i: (0, 0)),
        pl.BlockSpec((8, D), lambda i: (0, 0)),
    ],
    out_specs=[
        pl.BlockSpec((ROW_BLK, D), lambda i: (i, 0)),
        pl.BlockSpec((ROW_BLK, D), lambda i: (i, 0)),
    ],
    out_shape=[
        jax.ShapeDtypeStruct((NPAD, D), jnp.float32),
        jax.ShapeDtypeStruct((NPAD, D), jnp.float32),
    ],
)


def _tc_combine_body(p_ref, degp_ref, z1_ref, w_ref, b_ref, y_ref, z_ref):
    agg = p_ref[0].astype(jnp.float32) + p_ref[1].astype(jnp.float32)
    deg = jnp.maximum(jnp.sum(degp_ref[...], axis=0), 1.0)
    h = jnp.maximum(agg / deg[:, None] + z1_ref[...], 0.0)
    out = jnp.dot(h, w_ref[...],
                  preferred_element_type=jnp.float32,
                  precision=lax.Precision.HIGHEST)
    y_ref[...] = out[:, :D]
    z_ref[...] = out[:, D:] + b_ref[0:1, :]


_tc_combine = pl.pallas_call(
    _tc_combine_body,
    grid=(NPAD // ROW_BLK,),
    in_specs=[
        pl.BlockSpec((NC, ROW_BLK, D), lambda i: (0, i, 0)),
        pl.BlockSpec((NW, ROW_BLK), lambda i: (0, i)),
        pl.BlockSpec((ROW_BLK, D), lambda i: (i, 0)),
        pl.BlockSpec((D, 2 * D), lambda i: (0, 0)),
        pl.BlockSpec((8, D), lambda i: (0, 0)),
    ],
    out_specs=[
        pl.BlockSpec((ROW_BLK, D), lambda i: (i, 0)),
        pl.BlockSpec((ROW_BLK, D), lambda i: (i, 0)),
    ],
    out_shape=[
        jax.ShapeDtypeStruct((NPAD, D), jnp.float32),
        jax.ShapeDtypeStruct((NPAD, D), jnp.float32),
    ],
)


def _tc_final_body(p_ref, degp_ref, z2_ref, out_ref):
    agg = p_ref[0].astype(jnp.float32) + p_ref[1].astype(jnp.float32)
    deg = jnp.maximum(jnp.sum(degp_ref[...], axis=0), 1.0)
    out_ref[...] = agg / deg[:, None] + z2_ref[...]


_tc_final = pl.pallas_call(
    _tc_final_body,
    grid=(NPAD // ROW_BLK,),
    in_specs=[
        pl.BlockSpec((NC, ROW_BLK, D), lambda i: (0, i, 0)),
        pl.BlockSpec((NW, ROW_BLK), lambda i: (0, i)),
        pl.BlockSpec((ROW_BLK, D), lambda i: (i, 0)),
    ],
    out_specs=pl.BlockSpec((ROW_BLK, D), lambda i: (i, 0)),
    out_shape=jax.ShapeDtypeStruct((NPAD, D), jnp.float32),
)


@jax.jit
def kernel(x, edge_index, W1_l, b1, W1_r, W2_l, b2, W2_r):
    src = edge_index[0].astype(jnp.int32)
    dst = edge_index[1].astype(jnp.int32)
    src = jnp.concatenate([src, jnp.zeros((EPAD - E,), jnp.int32)])
    dst = jnp.concatenate([dst, jnp.full((EPAD - E,), NPAD - 1, jnp.int32)])
    eidx = jnp.stack([src.reshape(TOTCH, CH),
                      dst.reshape(TOTCH, CH)], axis=1)

    x_pad = jnp.pad(x, ((0, NPAD - N), (0, 0)))
    w1 = jnp.concatenate([W1_l, W1_r], axis=1)
    w2 = jnp.concatenate([W2_l, W2_r], axis=1)
    b1b = jnp.broadcast_to(b1.reshape(1, D), (8, D))
    b2b = jnp.broadcast_to(b2.reshape(1, D), (8, D))
    z2d = jnp.zeros((NPAD, D), jnp.float32)
    z1d = jnp.zeros((NPAD,), jnp.float32)

    y1, z1 = _tc_transform(x_pad, w1, b1b)
    parts1, degp = _make_sc_agg(True)(y1, eidx, z2d, z1d)
    y2, z2 = _tc_combine(parts1, degp, z1, w2, b2b)
    parts2, = _make_sc_agg(False)(y2, eidx, z2d, z1d)
    out = _tc_final(parts2, degp, z2)
    return out[:N]


# speedup vs baseline: 1.1626x; 1.1626x over previous
import functools

import jax
import jax.numpy as jnp
from jax import lax
from jax.experimental import pallas as pl
from jax.experimental.pallas import tpu as pltpu
from jax.experimental.pallas import tpu_sc as plsc

N = 10000
E = 320000
D = 128

NC = 2
NS = 16
NW = NC * NS

NPAD = 10240
RPW = NPAD // NS
CH = 64
NCHUNK = 160
EPW = NCHUNK * CH
EPAD = EPW * NW
NRB = 4
NIB = 8

ROW_BLK = 2560


def _sc_agg_body(with_deg, y_hbm, eidx_hbm, z2d_hbm, z1d_hbm,
                 parts_hbm, deg_hbm, acc, ibuf, rows, deg_l, sems):
    c = lax.axis_index("c")
    s = lax.axis_index("s")
    wid = s * NC + c
    gsem = sems[:NRB]
    isem = sems[NRB:]

    def issue_idx(g, slot):
        pltpu.async_copy(eidx_hbm.at[wid, g], ibuf.at[slot], isem[slot])

    def wait_idx(slot):
        pltpu.make_async_copy(eidx_hbm.at[0, 0], ibuf.at[slot],
                              isem[slot]).wait()

    def issue_gather(slot, b):
        pltpu.async_copy(y_hbm.at[ibuf.at[slot, 0]], rows.at[b], gsem[b])

    def wait_gather(b):
        pltpu.make_async_copy(y_hbm.at[pl.ds(0, CH)], rows.at[b],
                              gsem[b]).wait()

    def scatter(slot, b):
        pltpu.sync_copy(rows.at[b], acc.at[ibuf.at[slot, 1]], add=True)

    ones = jnp.full((16,), 1.0, jnp.float32)

    def deg_update(slot):
        if with_deg:
            for j in range(CH // 16):
                idx16 = ibuf[slot, 1, pl.ds(j * 16, 16)]
                plsc.addupdate_scatter(deg_l, [idx16], ones)

    for g in range(5):
        issue_idx(g, g)
    pltpu.sync_copy(z2d_hbm.at[pl.ds(s * RPW, RPW)], acc.at[pl.ds(s * RPW, RPW)])
    if with_deg:
        pltpu.sync_copy(z1d_hbm.at[:], deg_l)
    plsc.subcore_barrier()

    for g in range(3):
        wait_idx(g)
        issue_gather(g, g)

    def octet(k, carry):
        for u in range(NIB):
            g = NIB * k + u
            b = u % NRB
            wait_gather(b)

            @pl.when(g + 3 < NCHUNK)
            def _():
                wait_idx((u + 3) % NIB)
                issue_gather((u + 3) % NIB, (u + 3) % NRB)

            @pl.when(g + 5 < NCHUNK)
            def _():
                issue_idx(g + 5, (u + 5) % NIB)

            scatter(u, b)
            deg_update(u)
        return carry

    lax.fori_loop(0, NCHUNK // NIB, octet, 0)
    plsc.subcore_barrier()

    pltpu.sync_copy(acc.at[pl.ds(s * RPW, RPW)],
                    parts_hbm.at[c, pl.ds(s * RPW, RPW)])
    if with_deg:
        pltpu.sync_copy(deg_l, deg_hbm.at[wid])


def _make_sc_agg(with_deg):
    mesh = plsc.VectorSubcoreMesh(core_axis_name="c", subcore_axis_name="s",
                                  num_cores=NC, num_subcores=NS)
    out_type = [jax.ShapeDtypeStruct((NC, NPAD, D), jnp.float32)]
    if with_deg:
        out_type.append(jax.ShapeDtypeStruct((NW, NPAD), jnp.float32))
    nsem = NRB + NIB
    scratch = [
        pltpu.VMEM_SHARED((NPAD, D), jnp.float32),
        pltpu.VMEM((NIB, 2, CH), jnp.int32),
        pltpu.VMEM((NRB, CH, D), jnp.float32),
    ]
    if with_deg:
        scratch.append(pltpu.VMEM((NPAD,), jnp.float32))
    scratch += [pltpu.SemaphoreType.DMA] * nsem

    def body(y_hbm, eidx_hbm, z2d_hbm, z1d_hbm, *rest):
        if with_deg:
            parts_hbm, deg_hbm = rest[0], rest[1]
            acc, ibuf, rows, deg_l = rest[2:6]
            sems = rest[6:]
        else:
            parts_hbm, deg_hbm = rest[0], None
            acc, ibuf, rows = rest[1:4]
            deg_l = None
            sems = rest[4:]
        _sc_agg_body(with_deg, y_hbm, eidx_hbm, z2d_hbm, z1d_hbm,
                     parts_hbm, deg_hbm, acc, ibuf, rows, deg_l, sems)

    return pl.kernel(body, out_type=out_type, mesh=mesh, scratch_types=scratch,
                     compiler_params=pltpu.CompilerParams(
                         needs_layout_passes=False),
                     name="sc_agg_deg" if with_deg else "sc_agg")


_make_sc_agg = functools.lru_cache(maxsize=None)(_make_sc_agg)


def _tc_transform_body(x_ref, w_ref, b_ref, y_ref, z_ref):
    out = jnp.dot(x_ref[...], w_ref[...],
                  preferred_element_type=jnp.float32,
                  precision=lax.Precision.HIGHEST)
    y_ref[...] = out[:, :D]
    z_ref[...] = out[:, D:] + b_ref[0:1, :]


_tc_transform = pl.pallas_call(
    _tc_transform_body,
    grid=(NPAD // ROW_BLK,),
    in_specs=[
        pl.BlockSpec((ROW_BLK, D), lambda i: (i, 0)),
        pl.BlockSpec((D, 2 * D), lambda i: (0, 0)),
        pl.BlockSpec((8, D), lambda i: (0, 0)),
    ],
    out_specs=[
        pl.BlockSpec((ROW_BLK, D), lambda i: (i, 0)),
        pl.BlockSpec((ROW_BLK, D), lambda i: (i, 0)),
    ],
    out_shape=[
        jax.ShapeDtypeStruct((NPAD, D), jnp.float32),
        jax.ShapeDtypeStruct((NPAD, D), jnp.float32),
    ],
)


def _tc_combine_body(p_ref, degp_ref, z1_ref, w_ref, b_ref, y_ref, z_ref):
    agg = p_ref[0] + p_ref[1]
    deg = jnp.maximum(jnp.sum(degp_ref[...], axis=0), 1.0)
    h = jnp.maximum(agg / deg[:, None] + z1_ref[...], 0.0)
    out = jnp.dot(h, w_ref[...],
                  preferred_element_type=jnp.float32,
                  precision=lax.Precision.HIGHEST)
    y_ref[...] = out[:, :D]
    z_ref[...] = out[:, D:] + b_ref[0:1, :]


_tc_combine = pl.pallas_call(
    _tc_combine_body,
    grid=(NPAD // ROW_BLK,),
    in_specs=[
        pl.BlockSpec((NC, ROW_BLK, D), lambda i: (0, i, 0)),
        pl.BlockSpec((NW, ROW_BLK), lambda i: (0, i)),
        pl.BlockSpec((ROW_BLK, D), lambda i: (i, 0)),
        pl.BlockSpec((D, 2 * D), lambda i: (0, 0)),
        pl.BlockSpec((8, D), lambda i: (0, 0)),
    ],
    out_specs=[
        pl.BlockSpec((ROW_BLK, D), lambda i: (i, 0)),
        pl.BlockSpec((ROW_BLK, D), lambda i: (i, 0)),
    ],
    out_shape=[
        jax.ShapeDtypeStruct((NPAD, D), jnp.float32),
        jax.ShapeDtypeStruct((NPAD, D), jnp.float32),
    ],
)


def _tc_final_body(p_ref, degp_ref, z2_ref, out_ref):
    agg = p_ref[0] + p_ref[1]
    deg = jnp.maximum(jnp.sum(degp_ref[...], axis=0), 1.0)
    out_ref[...] = agg / deg[:, None] + z2_ref[...]


_tc_final = pl.pallas_call(
    _tc_final_body,
    grid=(NPAD // ROW_BLK,),
    in_specs=[
        pl.BlockSpec((NC, ROW_BLK, D), lambda i: (0, i, 0)),
        pl.BlockSpec((NW, ROW_BLK), lambda i: (0, i)),
        pl.BlockSpec((ROW_BLK, D), lambda i: (i, 0)),
    ],
    out_specs=pl.BlockSpec((ROW_BLK, D), lambda i: (i, 0)),
    out_shape=jax.ShapeDtypeStruct((NPAD, D), jnp.float32),
)


@jax.jit
def kernel(x, edge_index, W1_l, b1, W1_r, W2_l, b2, W2_r):
    src = edge_index[0].astype(jnp.int32)
    dst = edge_index[1].astype(jnp.int32)
    src = jnp.concatenate([src, jnp.zeros((EPAD - E,), jnp.int32)])
    dst = jnp.concatenate([dst, jnp.full((EPAD - E,), NPAD - 1, jnp.int32)])
    eidx = jnp.stack([src.reshape(NW, NCHUNK, CH),
                      dst.reshape(NW, NCHUNK, CH)], axis=2)

    x_pad = jnp.pad(x, ((0, NPAD - N), (0, 0)))
    w1 = jnp.concatenate([W1_l, W1_r], axis=1)
    w2 = jnp.concatenate([W2_l, W2_r], axis=1)
    b1b = jnp.broadcast_to(b1.reshape(1, D), (8, D))
    b2b = jnp.broadcast_to(b2.reshape(1, D), (8, D))
    z2d = jnp.zeros((NPAD, D), jnp.float32)
    z1d = jnp.zeros((NPAD,), jnp.float32)

    y1, z1 = _tc_transform(x_pad, w1, b1b)
    parts1, degp = _make_sc_agg(True)(y1, eidx, z2d, z1d)
    y2, z2 = _tc_combine(parts1, degp, z1, w2, b2b)
    parts2, = _make_sc_agg(False)(y2, eidx, z2d, z1d)
    out = _tc_final(parts2, degp, z2)
    return out[:N]
